# Initial kernel scaffold; baseline (speedup 1.0000x reference)
import jax
import jax.numpy as jnp
from jax.experimental import pallas as pl


def kernel(z, codebook):
    raise NotImplementedError("write your pallas kernel here")


# speedup vs baseline: 1.1102x; 1.1102x over previous
import functools

import jax
import jax.numpy as jnp
from jax import lax
from jax.experimental import pallas as pl
from jax.experimental.pallas import tpu as pltpu
from jax.experimental.pallas import tpu_sc as plsc

VOCAB = 8192
DIM = 32
BATCH = 8192
ROW_BLOCK = 256
GRID = BATCH // ROW_BLOCK


def _vq_body(z_ref, cbt_ref, idx_ref, counts_ref, loss_ref, perp_ref):
    i = pl.program_id(0)
    z_blk = z_ref[...]
    cbt = cbt_ref[...]

    zc = jnp.dot(z_blk, cbt, preferred_element_type=jnp.float32)
    znorm = jnp.sum(z_blk * z_blk, axis=1, keepdims=True)
    cnorm = jnp.sum(cbt * cbt, axis=0, keepdims=True)
    d = (znorm + cnorm) - 2.0 * zc

    half = VOCAB // 2
    d0 = d[:, :half]
    d1 = d[:, half:]
    kiota = lax.broadcasted_iota(jnp.int32, d0.shape, 1)
    dmin0 = jnp.min(d0, axis=1, keepdims=True)
    idx0 = jnp.min(jnp.where(d0 == dmin0, kiota, VOCAB), axis=1,
                   keepdims=True)
    dmin1 = jnp.min(d1, axis=1, keepdims=True)
    idx1 = jnp.min(jnp.where(d1 == dmin1, kiota + half, VOCAB), axis=1,
                   keepdims=True)
    v0 = dmin0.astype(jnp.bfloat16).astype(jnp.float32)
    win1 = dmin1 < v0
    idx = jnp.where(win1, idx1, idx0)
    dsel = jnp.where(win1, dmin1, dmin0)
    idx_ref[...] = idx

    kfull = lax.broadcasted_iota(jnp.int32, d.shape, 1)
    contrib = jnp.sum((kfull == idx).astype(jnp.float32), axis=0,
                      keepdims=True)
    loss_part = jnp.sum(dsel).reshape(1, 1)

    @pl.when(i == 0)
    def _init():
        counts_ref[...] = contrib
        loss_ref[...] = loss_part

    @pl.when(i > 0)
    def _acc():
        counts_ref[...] += contrib
        loss_ref[...] += loss_part

    @pl.when(i == GRID - 1)
    def _finish():
        p = counts_ref[...] * (1.0 / BATCH)
        ent = -jnp.sum(p * jnp.log(p + 1e-10))
        perp_ref[...] = jnp.exp(ent).reshape(1, 1)
        loss_ref[...] = loss_ref[...] * (1.0 / (BATCH * DIM))


def _tc_quantize(z, cbt):
    return pl.pallas_call(
        _vq_body,
        grid=(GRID,),
        in_specs=[
            pl.BlockSpec((ROW_BLOCK, DIM), lambda i: (i, 0)),
            pl.BlockSpec((DIM, VOCAB), lambda i: (0, 0)),
        ],
        out_specs=[
            pl.BlockSpec((ROW_BLOCK, 1), lambda i: (i, 0)),
            pl.BlockSpec((1, VOCAB), lambda i: (0, 0)),
            pl.BlockSpec((1, 1), lambda i: (0, 0)),
            pl.BlockSpec((1, 1), lambda i: (0, 0)),
        ],
        out_shape=[
            jax.ShapeDtypeStruct((BATCH, 1), jnp.int32),
            jax.ShapeDtypeStruct((1, VOCAB), jnp.float32),
            jax.ShapeDtypeStruct((1, 1), jnp.float32),
            jax.ShapeDtypeStruct((1, 1), jnp.float32),
        ],
        compiler_params=pltpu.CompilerParams(
            dimension_semantics=("arbitrary",)),
    )(z, cbt)


def _make_sc_gather():
    info = plsc.get_sparse_core_info()
    nw = info.num_cores * info.num_subcores
    b_per_w = BATCH // nw
    mesh = plsc.VectorSubcoreMesh(core_axis_name="c", subcore_axis_name="s")

    @functools.partial(
        pl.kernel,
        mesh=mesh,
        out_type=jax.ShapeDtypeStruct((BATCH, DIM), jnp.float32),
        scratch_types=[
            pltpu.VMEM((b_per_w,), jnp.int32),
            pltpu.VMEM((b_per_w, DIM), jnp.float32),
            pltpu.SemaphoreType.DMA,
        ],
        compiler_params=pltpu.CompilerParams(use_tc_tiling_on_sc=False),
    )
    def gather_k(table_hbm, idx_hbm, out_hbm, idx_v, rows_v, sem):
        wid = lax.axis_index("s") * info.num_cores + lax.axis_index("c")
        base = wid * b_per_w
        pltpu.sync_copy(idx_hbm.at[pl.ds(base, b_per_w)], idx_v)
        pltpu.async_copy(table_hbm.at[idx_v], rows_v, sem).wait()
        pltpu.sync_copy(rows_v, out_hbm.at[pl.ds(base, b_per_w)])

    return gather_k


def kernel(z, codebook):
    cbt = codebook.T
    idx2d, _counts, loss, perp = _tc_quantize(z, cbt)
    indices = idx2d[:, 0]
    z_q = _make_sc_gather()(codebook, indices)
    return z_q, indices, loss[0, 0], perp[0, 0]


# speedup vs baseline: 1.4246x; 1.2833x over previous
import functools

import jax
import jax.numpy as jnp
from jax import lax
from jax.experimental import pallas as pl
from jax.experimental.pallas import tpu as pltpu
from jax.experimental.pallas import tpu_sc as plsc

VOCAB = 8192
EMBED = 32
BATCH = 8192
ROW_BLOCK = 256
GRID = BATCH // ROW_BLOCK


def _vq_body(z_ref, cbt_ref, idx_ref, loss_ref):
    i = pl.program_id(0)
    z_blk = z_ref[...]
    cbt = cbt_ref[...]

    zc2 = jnp.dot(z_blk, cbt * 2.0, preferred_element_type=jnp.float32)
    znorm = jnp.sum(z_blk * z_blk, axis=1, keepdims=True)
    cnorm = jnp.sum(cbt * cbt, axis=0, keepdims=True)
    d = (znorm + cnorm) - zc2

    half = VOCAB // 2
    d0 = d[:, :half]
    d1 = d[:, half:]
    dmin0 = jnp.min(d0, axis=1, keepdims=True)
    idx0 = jnp.argmin(d0, axis=1).reshape(ROW_BLOCK, 1)
    dmin1 = jnp.min(d1, axis=1, keepdims=True)
    idx1 = jnp.argmin(d1, axis=1).reshape(ROW_BLOCK, 1) + half
    v0 = dmin0.astype(jnp.bfloat16).astype(jnp.float32)
    win1 = dmin1 < v0
    idx = jnp.where(win1, idx1, idx0)
    dsel = jnp.where(win1, dmin1, dmin0)
    idx_ref[...] = idx

    loss_part = jnp.sum(dsel).reshape(1, 1)

    @pl.when(i == 0)
    def _init():
        loss_ref[...] = loss_part

    @pl.when(i > 0)
    def _acc():
        loss_ref[...] += loss_part

    @pl.when(i == GRID - 1)
    def _finish():
        loss_ref[...] = loss_ref[...] * (1.0 / (BATCH * EMBED))


def _tc_quantize(z, cbt):
    return pl.pallas_call(
        _vq_body,
        grid=(GRID,),
        in_specs=[
            pl.BlockSpec((ROW_BLOCK, EMBED), lambda i: (i, 0)),
            pl.BlockSpec((EMBED, VOCAB), lambda i: (0, 0)),
        ],
        out_specs=[
            pl.BlockSpec((ROW_BLOCK, 1), lambda i: (i, 0)),
            pl.BlockSpec((1, 1), lambda i: (0, 0)),
        ],
        out_shape=[
            jax.ShapeDtypeStruct((BATCH, 1), jnp.int32),
            jax.ShapeDtypeStruct((1, 1), jnp.float32),
        ],
        compiler_params=pltpu.CompilerParams(
            dimension_semantics=("arbitrary",)),
    )(z, cbt)


def _entropy_body(h_ref, perp_ref):
    counts = jnp.sum(h_ref[...], axis=0, keepdims=True)
    p = counts * (1.0 / BATCH)
    ent = -jnp.sum(p * jnp.log(p + 1e-10))
    perp_ref[...] = jnp.exp(ent).reshape(1, 1)


def _tc_entropy(hists):
    return pl.pallas_call(
        _entropy_body,
        out_shape=jax.ShapeDtypeStruct((1, 1), jnp.float32),
    )(hists)


def _make_sc_gather():
    info = plsc.get_sparse_core_info()
    nw = info.num_cores * info.num_subcores
    b_per_w = BATCH // nw
    n_chunks = b_per_w // 16
    mesh = plsc.VectorSubcoreMesh(core_axis_name="c", subcore_axis_name="s")

    @functools.partial(
        pl.kernel,
        mesh=mesh,
        out_type=[
            jax.ShapeDtypeStruct((BATCH, EMBED), jnp.float32),
            jax.ShapeDtypeStruct((nw, VOCAB), jnp.float32),
        ],
        scratch_types=[
            pltpu.VMEM((b_per_w,), jnp.int32),
            pltpu.VMEM((b_per_w, EMBED), jnp.float32),
            pltpu.VMEM((VOCAB,), jnp.float32),
            pltpu.SemaphoreType.DMA,
        ],
        compiler_params=pltpu.CompilerParams(use_tc_tiling_on_sc=False,
                                             needs_layout_passes=False),
    )
    def gather_k(table_hbm, idx_hbm, out_hbm, hist_hbm,
                 idx_v, rows_v, hist_v, sem):
        wid = lax.axis_index("s") * info.num_cores + lax.axis_index("c")
        base = wid * b_per_w
        pltpu.sync_copy(idx_hbm.at[pl.ds(base, b_per_w)], idx_v)
        gather = pltpu.async_copy(table_hbm.at[idx_v], rows_v, sem)

        zeros16 = jnp.zeros((16,), jnp.float32)

        def _zero(j, carry):
            hist_v[pl.ds(j * 16, 16)] = zeros16
            return carry

        lax.fori_loop(0, VOCAB // 16, _zero, 0, unroll=8)

        def _hist(c, carry):
            iv = idx_v[pl.ds(c * 16, 16)]
            rc, last = plsc.scan_count(iv)
            plsc.addupdate_scatter(hist_v, [iv], rc.astype(jnp.float32),
                                   mask=last)
            return carry

        lax.fori_loop(0, n_chunks, _hist, 0, unroll=4)

        pltpu.sync_copy(hist_v, hist_hbm.at[wid])
        gather.wait()
        pltpu.sync_copy(rows_v, out_hbm.at[pl.ds(base, b_per_w)])

    return gather_k


def kernel(z, codebook):
    cbt = codebook.T
    idx2d, loss = _tc_quantize(z, cbt)
    indices = idx2d[:, 0]
    z_q, hists = _make_sc_gather()(codebook, indices)
    perp = _tc_entropy(hists)
    return z_q, indices, loss[0, 0], perp[0, 0]
